# baseline (speedup 1.0000x reference)
import functools

import jax
import jax.numpy as jnp
from jax import lax
from jax.experimental import pallas as pl
from jax.experimental.pallas import tpu as pltpu
from jax.experimental.pallas import tpu_sc as plsc

_L = 16
_NW = 32
_RR = 16


def _sc_body(img_h, mask_h, boost_h, ab_h, out_h, img_v, mask_v, boost_v,
             out_v, ab_v, si0, si1, so0, so1, rows_half, W):
    cid = lax.axis_index("c")
    sid = lax.axis_index("s")
    w = sid * 2 + cid

    pltpu.sync_copy(ab_h, ab_v)
    a = ab_v[pl.ds(0, _L)]
    b = ab_v[pl.ds(_L, _L)]

    n_chunks = rows_half // _RR
    sem_in = (si0, si1)
    sem_out = (so0, so1)
    n_vec = W // _L

    def in_copies(ir, mr, b_):
        return (
            pltpu.make_async_copy(img_h.at[pl.ds(ir, _RR)], img_v.at[b_],
                                  sem_in[b_]),
            pltpu.make_async_copy(mask_h.at[pl.ds(mr, _RR)], mask_v.at[b_],
                                  sem_in[b_]),
            pltpu.make_async_copy(boost_h.at[pl.ds(ir, _RR)], boost_v.at[b_],
                                  sem_in[b_]),
        )

    for t in range(3):
        h = w + t * _NW
        p = h // 2
        img_base = p * (2 * rows_half) + (h % 2) * rows_half
        mask_base = (p // 3) * (2 * rows_half) + (h % 2) * rows_half

        for cp in in_copies(img_base, mask_base, 0):
            cp.start()

        def k_body(k, _, img_base=img_base, mask_base=mask_base):
            for b_ in (0, 1):
                c = 2 * k + b_
                ir = img_base + c * _RR
                mr = mask_base + c * _RR
                for cp in in_copies(ir, mr, b_):
                    cp.wait()

                @pl.when(c + 1 < n_chunks)
                def _():
                    for cp in in_copies(ir + _RR, mr + _RR, 1 - b_):
                        cp.start()

                @pl.when(c >= 2)
                def _():
                    pltpu.make_async_copy(out_v.at[b_],
                                          out_h.at[pl.ds(ir, _RR)],
                                          sem_out[b_]).wait()

                ib, mb, bs, ob = (img_v.at[b_], mask_v.at[b_],
                                  boost_v.at[b_], out_v.at[b_])

                @plsc.parallel_loop(0, _RR, 1, unroll=2)
                def _(r):
                    for cc in range(n_vec):
                        s = pl.ds(cc * _L, _L)
                        bt = bs[r, s] * a + b
                        ob[r, s] = jnp.where(mb[r, s] < 0.5, bt, ib[r, s])

                pltpu.make_async_copy(out_v.at[b_],
                                      out_h.at[pl.ds(ir, _RR)],
                                      sem_out[b_]).start()
            return 0

        lax.fori_loop(0, n_chunks // 2, k_body, 0)

        for b_ in (0, 1):
            c = n_chunks - 2 + b_
            ir = img_base + c * _RR
            pltpu.make_async_copy(out_v.at[b_], out_h.at[pl.ds(ir, _RR)],
                                  sem_out[b_]).wait()


def kernel(standardization, batchimg, batchmask, boost):
    batchimg = batchimg.astype(jnp.float32)
    batchmask = batchmask.astype(jnp.float32)
    boost = boost.astype(jnp.float32)
    B, C, H, W = batchimg.shape
    std = jnp.asarray(standardization)
    a = jnp.where(std != 0, jnp.float32(1.0 / 3.9), jnp.float32(0.5 / 3.9))
    b = jnp.where(std != 0, jnp.float32(0.0), jnp.float32(0.5))
    ab = jnp.concatenate([jnp.full((_L,), a, jnp.float32),
                          jnp.full((_L,), b, jnp.float32)])

    rows_half = H // 2

    f = functools.partial(
        pl.kernel,
        out_type=jax.ShapeDtypeStruct((B * C * H, W), jnp.float32),
        mesh=plsc.VectorSubcoreMesh(core_axis_name="c", subcore_axis_name="s"),
        compiler_params=pltpu.CompilerParams(use_tc_tiling_on_sc=True),
        scratch_types=[
            pltpu.VMEM((2, _RR, W), jnp.float32),
            pltpu.VMEM((2, _RR, W), jnp.float32),
            pltpu.VMEM((2, _RR, W), jnp.float32),
            pltpu.VMEM((2, _RR, W), jnp.float32),
            pltpu.VMEM((2 * _L,), jnp.float32),
            pltpu.SemaphoreType.DMA,
            pltpu.SemaphoreType.DMA,
            pltpu.SemaphoreType.DMA,
            pltpu.SemaphoreType.DMA,
        ],
    )(functools.partial(_sc_body, rows_half=rows_half, W=W))
    out = f(batchimg.reshape(B * C * H, W), batchmask.reshape(B * H, W),
            boost.reshape(B * C * H, W), ab)
    return out.reshape(B, C, H, W)


# speedup vs baseline: 1.7633x; 1.7633x over previous
import functools

import jax
import jax.numpy as jnp
from jax import lax
from jax.experimental import pallas as pl
from jax.experimental.pallas import tpu as pltpu
from jax.experimental.pallas import tpu_sc as plsc

_L = 16
_NW = 32
_RR = 16
_STRIP = 64
_B_TC = 12


def _tc_body(ab_ref, img_ref, mask_ref, boost_ref, out_ref):
    a = ab_ref[0]
    b = ab_ref[1]
    m = mask_ref[...]
    bt = boost_ref[...] * a + b
    out_ref[...] = jnp.where(m[:, None, :, :] < 0.5, bt, img_ref[...])


def _sc_body(img_h, mask_h, boost_h, ab_h, out_h, img_v, mask_v, boost_v,
             out_v, ab_v, si0, si1, so0, so1, plane0, strips_per_worker,
             H, W):
    cid = lax.axis_index("c")
    sid = lax.axis_index("s")
    w = sid * 2 + cid

    pltpu.sync_copy(ab_h, ab_v)
    a = ab_v[pl.ds(0, _L)]
    b = ab_v[pl.ds(_L, _L)]

    n_chunks = _STRIP // _RR
    strips_per_plane = H // _STRIP
    sem_in = (si0, si1)
    sem_out = (so0, so1)
    n_vec = W // _L

    def in_copies(ir, mr, b_):
        return (
            pltpu.make_async_copy(img_h.at[pl.ds(ir, _RR)], img_v.at[b_],
                                  sem_in[b_]),
            pltpu.make_async_copy(mask_h.at[pl.ds(mr, _RR)], mask_v.at[b_],
                                  sem_in[b_]),
            pltpu.make_async_copy(boost_h.at[pl.ds(ir, _RR)], boost_v.at[b_],
                                  sem_in[b_]),
        )

    for t in range(strips_per_worker):
        u = w + t * _NW
        p_local = u // strips_per_plane
        row_in_plane = (u % strips_per_plane) * _STRIP
        p = plane0 + p_local
        img_base = p * H + row_in_plane
        mask_base = (p // 3) * H + row_in_plane
        out_base = p_local * H + row_in_plane

        for cp in in_copies(img_base, mask_base, 0):
            cp.start()

        def k_body(k, _, img_base=img_base, mask_base=mask_base,
                   out_base=out_base):
            for b_ in (0, 1):
                c = 2 * k + b_
                ir = img_base + c * _RR
                mr = mask_base + c * _RR
                orow = out_base + c * _RR
                for cp in in_copies(ir, mr, b_):
                    cp.wait()

                @pl.when(c + 1 < n_chunks)
                def _():
                    for cp in in_copies(ir + _RR, mr + _RR, 1 - b_):
                        cp.start()

                @pl.when(c >= 2)
                def _():
                    pltpu.make_async_copy(out_v.at[b_],
                                          out_h.at[pl.ds(orow, _RR)],
                                          sem_out[b_]).wait()

                ib, mb, bs, ob = (img_v.at[b_], mask_v.at[b_],
                                  boost_v.at[b_], out_v.at[b_])

                def row_body(r, _):
                    for cc in range(n_vec):
                        s = pl.ds(cc * _L, _L)
                        bt = bs[r, s] * a + b
                        ob[r, s] = jnp.where(mb[r, s] < 0.5, bt, ib[r, s])
                    return 0

                lax.fori_loop(0, _RR, row_body, 0)

                pltpu.make_async_copy(out_v.at[b_],
                                      out_h.at[pl.ds(orow, _RR)],
                                      sem_out[b_]).start()
            return 0

        lax.fori_loop(0, n_chunks // 2, k_body, 0)

        for b_ in (0, 1):
            c = n_chunks - 2 + b_
            orow = out_base + c * _RR
            pltpu.make_async_copy(out_v.at[b_], out_h.at[pl.ds(orow, _RR)],
                                  sem_out[b_]).wait()


def kernel(standardization, batchimg, batchmask, boost):
    batchimg = batchimg.astype(jnp.float32)
    batchmask = batchmask.astype(jnp.float32)
    boost = boost.astype(jnp.float32)
    B, C, H, W = batchimg.shape
    std = jnp.asarray(standardization)
    a = jnp.where(std != 0, jnp.float32(1.0 / 3.9), jnp.float32(0.5 / 3.9))
    b = jnp.where(std != 0, jnp.float32(0.0), jnp.float32(0.5))
    ab_tc = jnp.stack([a, b]).astype(jnp.float32)
    ab_sc = jnp.concatenate([jnp.full((_L,), a, jnp.float32),
                             jnp.full((_L,), b, jnp.float32)])

    b_sc = B - _B_TC
    n_strips = b_sc * C * (H // _STRIP)
    strips_per_worker = n_strips // _NW

    tc_out = pl.pallas_call(
        _tc_body,
        grid=(_B_TC,),
        compiler_params=pltpu.CompilerParams(
            dimension_semantics=("parallel",),
        ),
        in_specs=[
            pl.BlockSpec(memory_space=pltpu.SMEM),
            pl.BlockSpec((1, C, H, W), lambda i: (i, 0, 0, 0)),
            pl.BlockSpec((1, H, W), lambda i: (i, 0, 0)),
            pl.BlockSpec((1, C, H, W), lambda i: (i, 0, 0, 0)),
        ],
        out_specs=pl.BlockSpec((1, C, H, W), lambda i: (i, 0, 0, 0)),
        out_shape=jax.ShapeDtypeStruct((B, C, H, W), jnp.float32),
    )(ab_tc, batchimg, batchmask, boost)

    sc_f = functools.partial(
        pl.kernel,
        out_type=jax.ShapeDtypeStruct((b_sc * C * H, W), jnp.float32),
        mesh=plsc.VectorSubcoreMesh(core_axis_name="c", subcore_axis_name="s"),
        compiler_params=pltpu.CompilerParams(use_tc_tiling_on_sc=True),
        scratch_types=[
            pltpu.VMEM((2, _RR, W), jnp.float32),
            pltpu.VMEM((2, _RR, W), jnp.float32),
            pltpu.VMEM((2, _RR, W), jnp.float32),
            pltpu.VMEM((2, _RR, W), jnp.float32),
            pltpu.VMEM((2 * _L,), jnp.float32),
            pltpu.SemaphoreType.DMA,
            pltpu.SemaphoreType.DMA,
            pltpu.SemaphoreType.DMA,
            pltpu.SemaphoreType.DMA,
        ],
    )(functools.partial(_sc_body, plane0=_B_TC * C,
                        strips_per_worker=strips_per_worker, H=H, W=W))
    sc_out = sc_f(batchimg.reshape(B * C * H, W),
                  batchmask.reshape(B * H, W),
                  boost.reshape(B * C * H, W), ab_sc)

    return lax.dynamic_update_slice(
        tc_out, sc_out.reshape(b_sc, C, H, W), (_B_TC, 0, 0, 0))


# speedup vs baseline: 2.8037x; 1.5900x over previous
import jax
import jax.numpy as jnp
from jax.experimental import pallas as pl
from jax.experimental.pallas import tpu as pltpu


def _select_kernel(ab_ref, img_ref, mask_ref, boost_ref, out_ref):
    a = ab_ref[0]
    b = ab_ref[1]
    m = mask_ref[...]
    bt = boost_ref[...] * a + b
    out_ref[...] = jnp.where(m[:, None, :, :] < 0.5, bt, img_ref[...])


def kernel(standardization, batchimg, batchmask, boost):
    batchimg = batchimg.astype(jnp.float32)
    batchmask = batchmask.astype(jnp.float32)
    boost = boost.astype(jnp.float32)
    B, C, H, W = batchimg.shape
    std = jnp.asarray(standardization)
    a = jnp.where(std != 0, jnp.float32(1.0 / 3.9), jnp.float32(0.5 / 3.9))
    b = jnp.where(std != 0, jnp.float32(0.0), jnp.float32(0.5))
    ab = jnp.stack([a, b]).astype(jnp.float32)

    out = pl.pallas_call(
        _select_kernel,
        grid=(B,),
        compiler_params=pltpu.CompilerParams(
            dimension_semantics=("parallel",),
        ),
        in_specs=[
            pl.BlockSpec(memory_space=pltpu.SMEM),
            pl.BlockSpec((1, C, H, W), lambda i: (i, 0, 0, 0)),
            pl.BlockSpec((1, H, W), lambda i: (i, 0, 0)),
            pl.BlockSpec((1, C, H, W), lambda i: (i, 0, 0, 0)),
        ],
        out_specs=pl.BlockSpec((1, C, H, W), lambda i: (i, 0, 0, 0)),
        out_shape=jax.ShapeDtypeStruct((B, C, H, W), jnp.float32),
    )(ab, batchimg, batchmask, boost)
    return out


# speedup vs baseline: 2.8086x; 1.0017x over previous
import jax
import jax.numpy as jnp
from jax.experimental import pallas as pl
from jax.experimental.pallas import tpu as pltpu


def _select_kernel(ab_ref, img_ref, mask_ref, boost_ref, out_ref):
    a = ab_ref[0]
    b = ab_ref[1]
    m = mask_ref[...]
    bt = boost_ref[...] * a + b
    out_ref[...] = jnp.where(m[:, None, :, :] < 0.5, bt, img_ref[...])


def kernel(standardization, batchimg, batchmask, boost):
    batchimg = batchimg.astype(jnp.float32)
    batchmask = batchmask.astype(jnp.float32)
    boost = boost.astype(jnp.float32)
    B, C, H, W = batchimg.shape
    std = jnp.asarray(standardization)
    a = jnp.where(std != 0, jnp.float32(1.0 / 3.9), jnp.float32(0.5 / 3.9))
    b = jnp.where(std != 0, jnp.float32(0.0), jnp.float32(0.5))
    ab = jnp.stack([a, b]).astype(jnp.float32)

    out = pl.pallas_call(
        _select_kernel,
        grid=(B,),
        compiler_params=pltpu.CompilerParams(
            dimension_semantics=("arbitrary",),
        ),
        in_specs=[
            pl.BlockSpec(memory_space=pltpu.SMEM),
            pl.BlockSpec((1, C, H, W), lambda i: (i, 0, 0, 0)),
            pl.BlockSpec((1, H, W), lambda i: (i, 0, 0)),
            pl.BlockSpec((1, C, H, W), lambda i: (i, 0, 0, 0)),
        ],
        out_specs=pl.BlockSpec((1, C, H, W), lambda i: (i, 0, 0, 0)),
        out_shape=jax.ShapeDtypeStruct((B, C, H, W), jnp.float32),
    )(ab, batchimg, batchmask, boost)
    return out
